# baseline (speedup 1.0000x reference)
import functools

import jax
import jax.numpy as jnp
from jax import lax
from jax.experimental import pallas as pl
from jax.experimental.pallas import tpu as pltpu
from jax.experimental.pallas import tpu_sc as plsc

_NC = 2
_NS = 16
_LANES = 16
_NW = _NC * _NS

_N = 100000
_CHUNK = 3136
_CHUNK_LAST = _N - 31 * _CHUNK
_N_SP = 119
_TBL_PAD = 128


@functools.partial(
    pl.kernel,
    out_type=jax.ShapeDtypeStruct((_N,), jnp.int32),
    mesh=plsc.VectorSubcoreMesh(core_axis_name="c", subcore_axis_name="s"),
    scratch_types=[
        pltpu.VMEM((_CHUNK,), jnp.int32),
        pltpu.VMEM((_TBL_PAD,), jnp.int32),
        pltpu.VMEM((_TBL_PAD,), jnp.int32),
        pltpu.VMEM((_CHUNK,), jnp.int32),
        pltpu.SemaphoreType.DMA,
    ],
    compiler_params=pltpu.CompilerParams(
        needs_layout_passes=False, skip_device_barrier=True),
)
def _species_lookup(an_hbm, tbl_hbm, out_hbm, an_v, tbl_v, inv_v, out_v, sem):
    wid = lax.axis_index("s") * _NC + lax.axis_index("c")
    base = wid * _CHUNK

    def run(ch):
        copy = pltpu.make_async_copy(
            an_hbm.at[pl.ds(base, ch)], an_v.at[pl.ds(0, ch)], sem)
        copy.start()

        pltpu.sync_copy(tbl_hbm, tbl_v.at[pl.ds(0, _N_SP)])
        for j in range(_TBL_PAD // _LANES):
            vals = tbl_v[pl.ds(j * _LANES, _LANES)]
            ids = lax.iota(jnp.int32, _LANES) + j * _LANES
            mask = None
            if (j + 1) * _LANES > _N_SP:
                mask = ids < _N_SP
            plsc.store_scatter(inv_v, [vals], ids, mask=mask)

        copy.wait()

        for i in range(ch // _LANES):
            a = an_v[pl.ds(i * _LANES, _LANES)]
            out_v[pl.ds(i * _LANES, _LANES)] = plsc.load_gather(inv_v, [a])

        pltpu.sync_copy(out_v.at[pl.ds(0, ch)], out_hbm.at[pl.ds(base, ch)])

    @pl.when(wid < _NW - 1)
    def _():
        run(_CHUNK)

    @pl.when(wid == _NW - 1)
    def _():
        run(_CHUNK_LAST)


def kernel(atomic_numbers, x, species_table):
    species = _species_lookup(atomic_numbers, species_table)
    return (species, x, atomic_numbers)


# speedup vs baseline: 1.0250x; 1.0250x over previous
import functools

import jax
import jax.numpy as jnp
from jax import lax
from jax.experimental import pallas as pl
from jax.experimental.pallas import tpu as pltpu
from jax.experimental.pallas import tpu_sc as plsc

_NC = 2
_NS = 16
_LANES = 16
_NW = _NC * _NS

_N = 100000
_CHUNK = 3136
_CHUNK_LAST = _N - 31 * _CHUNK
_N_SP = 119
_TBL_PAD = 128


@functools.partial(
    pl.kernel,
    out_type=jax.ShapeDtypeStruct((_N,), jnp.int32),
    mesh=plsc.VectorSubcoreMesh(core_axis_name="c", subcore_axis_name="s"),
    scratch_types=[
        pltpu.VMEM((_CHUNK,), jnp.int32),
        pltpu.VMEM((_TBL_PAD,), jnp.int32),
        pltpu.VMEM((_TBL_PAD,), jnp.int32),
        pltpu.VMEM((_CHUNK,), jnp.int32),
        pltpu.SemaphoreType.DMA,
    ],
    compiler_params=pltpu.CompilerParams(
        needs_layout_passes=False, skip_device_barrier=True),
    cost_estimate=pl.CostEstimate(
        flops=12_000_000, transcendentals=0, bytes_accessed=24_000_000),
)
def _species_lookup(an_hbm, tbl_hbm, out_hbm, an_v, tbl_v, inv_v, out_v, sem):
    wid = lax.axis_index("s") * _NC + lax.axis_index("c")
    base = wid * _CHUNK
    last = wid == _NW - 1

    @pl.when(~last)
    def _():
        pltpu.make_async_copy(
            an_hbm.at[pl.ds(base, _CHUNK)], an_v, sem).start()

    @pl.when(last)
    def _():
        pltpu.make_async_copy(
            an_hbm.at[pl.ds(base, _CHUNK_LAST)],
            an_v.at[pl.ds(0, _CHUNK_LAST)], sem).start()

    pltpu.sync_copy(tbl_hbm, tbl_v.at[pl.ds(0, _N_SP)])
    for j in range(_TBL_PAD // _LANES):
        vals = tbl_v[pl.ds(j * _LANES, _LANES)]
        ids = lax.iota(jnp.int32, _LANES) + j * _LANES
        mask = None
        if (j + 1) * _LANES > _N_SP:
            mask = ids < _N_SP
        plsc.store_scatter(inv_v, [vals], ids, mask=mask)

    @pl.when(~last)
    def _():
        pltpu.make_async_copy(
            an_hbm.at[pl.ds(base, _CHUNK)], an_v, sem).wait()

    @pl.when(last)
    def _():
        pltpu.make_async_copy(
            an_hbm.at[pl.ds(base, _CHUNK_LAST)],
            an_v.at[pl.ds(0, _CHUNK_LAST)], sem).wait()

    for i in range(_CHUNK // _LANES):
        a = an_v[pl.ds(i * _LANES, _LANES)]
        a = jnp.minimum(jnp.maximum(a, 0), _TBL_PAD - 1)
        out_v[pl.ds(i * _LANES, _LANES)] = plsc.load_gather(inv_v, [a])

    @pl.when(~last)
    def _():
        pltpu.sync_copy(out_v, out_hbm.at[pl.ds(base, _CHUNK)])

    @pl.when(last)
    def _():
        pltpu.sync_copy(out_v.at[pl.ds(0, _CHUNK_LAST)],
                        out_hbm.at[pl.ds(base, _CHUNK_LAST)])


def kernel(atomic_numbers, x, species_table):
    species = _species_lookup(atomic_numbers, species_table)
    return (species, x, atomic_numbers)
